# Initial kernel scaffold; baseline (speedup 1.0000x reference)
import jax
import jax.numpy as jnp
from jax.experimental import pallas as pl


def kernel(x, weight):
    raise NotImplementedError("write your pallas kernel here")


# speedup vs baseline: 1.8461x; 1.8461x over previous
import functools

import jax
import jax.numpy as jnp
from jax import lax
from jax.experimental import pallas as pl
from jax.experimental.pallas import tpu as pltpu
from jax.experimental.pallas import tpu_sc as plsc

NC, NS = 2, 16
NW = NC * NS
GATHER = 128
K = 8
CHUNK = K * GATHER


def _body(table, idx, out, idx_v, rows_v, sem):
    wid = lax.axis_index("s") * NC + lax.axis_index("c")
    rows_total = out.shape[0]
    b_per_w = rows_total // NW
    steps = b_per_w // CHUNK
    base_row = wid * b_per_w
    base_blk = wid * (b_per_w // GATHER)

    def step(g, _):
        pltpu.sync_copy(idx.at[pl.ds(base_blk + g * K, K)], idx_v)
        copies = [
            pltpu.async_copy(
                table.at[idx_v.at[j]],
                rows_v.at[pl.ds(j * GATHER, GATHER)],
                sem,
            )
            for j in range(K)
        ]
        for c in copies:
            c.wait()
        pltpu.sync_copy(rows_v, out.at[pl.ds(base_row + g * CHUNK, CHUNK)])
        return _

    lax.fori_loop(0, steps, step, None)


def kernel(x, weight):
    B, H = x.shape
    V, D = weight.shape
    rows = B * H
    idx2d = x.reshape(rows // GATHER, GATHER).astype(jnp.int32)

    mesh = plsc.VectorSubcoreMesh(
        core_axis_name="c", subcore_axis_name="s",
        num_cores=NC, num_subcores=NS,
    )
    run = pl.kernel(
        _body,
        out_type=jax.ShapeDtypeStruct((rows, D), jnp.float32),
        mesh=mesh,
        scratch_types=[
            pltpu.VMEM((K, GATHER), jnp.int32),
            pltpu.VMEM((CHUNK, D), jnp.float32),
            pltpu.SemaphoreType.DMA,
        ],
        compiler_params=pltpu.CompilerParams(use_tc_tiling_on_sc=False),
    )
    out = run(weight, idx2d)
    return out.reshape(B, H, D)


# speedup vs baseline: 1.8766x; 1.0165x over previous
import functools

import jax
import jax.numpy as jnp
from jax import lax
from jax.experimental import pallas as pl
from jax.experimental.pallas import tpu as pltpu
from jax.experimental.pallas import tpu_sc as plsc

NC, NS = 2, 16
NW = NC * NS
GATHER = 128
K = 4
CHUNK = K * GATHER


def _body(table, idx, out, idx_v, rows_v, gsem, ssem):
    wid = lax.axis_index("s") * NC + lax.axis_index("c")
    rows_total = out.shape[0]
    b_per_w = rows_total // NW
    chunks = b_per_w // CHUNK
    base_row = wid * b_per_w

    pltpu.sync_copy(idx.at[pl.ds(wid * chunks, chunks)], idx_v)

    def fire_gathers(c, b):
        for j in range(K):
            pltpu.async_copy(
                table.at[idx_v.at[c, j]],
                rows_v.at[b, pl.ds(j * GATHER, GATHER)],
                gsem.at[b],
            )

    def drain_gathers(b):
        pltpu.make_async_copy(
            out.at[pl.ds(0, CHUNK)], rows_v.at[b], gsem.at[b]
        ).wait()

    def fire_store(c, b):
        pltpu.async_copy(
            rows_v.at[b],
            out.at[pl.ds(base_row + c * CHUNK, CHUNK)],
            ssem.at[b],
        )

    def drain_store(b):
        pltpu.make_async_copy(
            rows_v.at[b], out.at[pl.ds(0, CHUNK)], ssem.at[b]
        ).wait()

    fire_gathers(0, 0)

    @pl.loop(0, chunks - 1)
    def _pipe(c):
        b = c % 2
        nb = 1 - b
        @pl.when(c >= 1)
        def _():
            drain_store(nb)
        fire_gathers(c + 1, nb)
        drain_gathers(b)
        fire_store(c, b)

    last = chunks - 1
    lb = last % 2
    drain_gathers(lb)
    fire_store(last, lb)
    drain_store(lb)
    drain_store(1 - lb)


def kernel(x, weight):
    B, H = x.shape
    V, D = weight.shape
    rows = B * H
    idx3d = x.reshape(rows // CHUNK, K, GATHER).astype(jnp.int32)
    chunks_per_w = rows // NW // CHUNK

    mesh = plsc.VectorSubcoreMesh(
        core_axis_name="c", subcore_axis_name="s",
        num_cores=NC, num_subcores=NS,
    )
    run = pl.kernel(
        _body,
        out_type=jax.ShapeDtypeStruct((rows, D), jnp.float32),
        mesh=mesh,
        scratch_types=[
            pltpu.VMEM((chunks_per_w, K, GATHER), jnp.int32),
            pltpu.VMEM((2, CHUNK, D), jnp.float32),
            pltpu.SemaphoreType.DMA((2,)),
            pltpu.SemaphoreType.DMA((2,)),
        ],
        compiler_params=pltpu.CompilerParams(use_tc_tiling_on_sc=False),
    )
    out = run(weight, idx3d)
    return out.reshape(B, H, D)
